# baseline (speedup 1.0000x reference)
import functools

import jax
import jax.numpy as jnp
from jax import lax
from jax.experimental import pallas as pl
from jax.experimental.pallas import tpu as pltpu
from jax.experimental.pallas import tpu_sc as plsc

_N = 65536
_H = 64
_D = 64
_K = 1024
_EPS = 1e-5

_SLABS = 4
_BN = 4096
_SC_CORES = 2
_SC_SUBCORES = 16
_NW = _SC_CORES * _SC_SUBCORES
_CHUNK = 512


def _tc_body(xy_ref, w1_ref, b1_ref, g_ref, be_ref, w2_ref, b2_ref, cbt_ref,
             cbtbf_ref, ks_ref, idx_ref, loss_ref, c2_ref):
    i = pl.program_id(0)

    @pl.when(i == 0)
    def _precompute():
        cbt_full = cbt_ref[...]
        c2_ref[...] = jnp.sum(cbt_full * cbt_full, axis=0, keepdims=True)

    bf = jnp.bfloat16
    xyf = xy_ref[...].astype(jnp.float32)
    nxy = xyf / 511.0 * 2.0 - 1.0
    h = jnp.dot(nxy.astype(bf), w1_ref[...].astype(bf),
                preferred_element_type=jnp.float32) + b1_ref[...]
    mu = jnp.mean(h, axis=-1, keepdims=True)
    var = jnp.mean((h - mu) ** 2, axis=-1, keepdims=True)
    h = (h - mu) / jnp.sqrt(var + _EPS) * g_ref[...] + be_ref[...]
    h = jnp.maximum(h, 0.0)
    z = jnp.dot(h.astype(bf), w2_ref[...].astype(bf),
                preferred_element_type=jnp.float32) + b2_ref[...]

    z2 = jnp.sum(z * z, axis=1, keepdims=True)
    c2 = c2_ref[...]
    zc2 = jnp.dot((2.0 * z).astype(bf), cbtbf_ref[...],
                  preferred_element_type=jnp.float32)
    d = z2 - zc2 + c2
    dmin = jnp.min(d, axis=1, keepdims=True)
    kf = jnp.where(d == dmin, ks_ref[...], float(_K))
    imin = jnp.min(kf, axis=1, keepdims=True).astype(jnp.int32)
    idx_ref[...] = imin

    @pl.when(i == 0)
    def _init():
        loss_ref[...] = jnp.zeros((1, 1), jnp.float32)

    loss_ref[...] += jnp.sum(dmin).reshape(1, 1)

    @pl.when(i == pl.num_programs(0) - 1)
    def _finish():
        loss_ref[...] = loss_ref[...] * (1.25 / (_N * _D))


def _tc_quantize(xy, W1, b1, gamma, beta, W2, b2, cbT, cbT_bf, ks):
    n = xy.shape[0]
    rep = lambda i: (0, 0)
    return pl.pallas_call(
        _tc_body,
        grid=(n // _BN,),
        in_specs=[
            pl.BlockSpec((_BN, 2), lambda i: (i, 0)),
            pl.BlockSpec((2, _H), rep),
            pl.BlockSpec((1, _H), rep),
            pl.BlockSpec((1, _H), rep),
            pl.BlockSpec((1, _H), rep),
            pl.BlockSpec((_H, _D), rep),
            pl.BlockSpec((1, _D), rep),
            pl.BlockSpec((_D, _K), rep),
            pl.BlockSpec((_D, _K), rep),
            pl.BlockSpec((1, _K), rep),
        ],
        out_specs=[
            pl.BlockSpec((_BN, 1), lambda i: (i, 0)),
            pl.BlockSpec((1, 1), rep),
        ],
        out_shape=[
            jax.ShapeDtypeStruct((n, 1), jnp.int32),
            jax.ShapeDtypeStruct((1, 1), jnp.float32),
        ],
        scratch_shapes=[pltpu.VMEM((1, _K), jnp.float32)],
        compiler_params=pltpu.CompilerParams(
            dimension_semantics=("arbitrary",)),
    )(xy, W1, b1, gamma, beta, W2, b2, cbT, cbT_bf, ks)


def _sc_gather(codebook, idx_flat):
    n = idx_flat.shape[0]
    rows_per_w = n // _NW
    mesh = plsc.VectorSubcoreMesh(core_axis_name="c", subcore_axis_name="s")

    @functools.partial(
        pl.kernel,
        mesh=mesh,
        out_type=jax.ShapeDtypeStruct((n, _D), jnp.float32),
        scratch_types=[
            pltpu.VMEM((_K, _D), jnp.float32),
            pltpu.VMEM_SHARED((_K, _D), jnp.float32),
            pltpu.VMEM((_CHUNK,), jnp.int32),
            pltpu.VMEM((_CHUNK, _D), jnp.float32),
            pltpu.SemaphoreType.DMA,
        ],
        compiler_params=pltpu.CompilerParams(use_tc_tiling_on_sc=False),
    )
    def gather_kernel(cb_hbm, idx_hbm, out_hbm, tmp_v, cb_sh, idx_v, rows_v,
                      sem):
        sid = lax.axis_index("s")
        wid = sid * _SC_CORES + lax.axis_index("c")

        @pl.when(sid == 0)
        def _stage():
            pltpu.sync_copy(cb_hbm, tmp_v)
            pltpu.sync_copy(tmp_v, cb_sh)

        plsc.subcore_barrier()
        base = wid * rows_per_w
        for c in range(rows_per_w // _CHUNK):
            off = base + c * _CHUNK
            pltpu.sync_copy(idx_hbm.at[pl.ds(off, _CHUNK)], idx_v)
            pltpu.async_copy(cb_sh.at[idx_v], rows_v, sem).wait()
            pltpu.sync_copy(rows_v, out_hbm.at[pl.ds(off, _CHUNK)])

    return gather_kernel(codebook, idx_flat)


def kernel(xy, W1, b1, gamma, beta, W2, b2, codebook):
    cbT = codebook.T
    cbT_bf = cbT.astype(jnp.bfloat16)
    ks = jnp.arange(_K, dtype=jnp.float32).reshape(1, _K)
    b1r, gr, ber = (b1.reshape(1, _H), gamma.reshape(1, _H),
                    beta.reshape(1, _H))
    b2r = b2.reshape(1, _D)

    slab = _N // _SLABS
    qs, idxs, losses = [], [], []
    for s in range(_SLABS):
        xy_s = lax.slice_in_dim(xy, s * slab, (s + 1) * slab, axis=0)
        idx_s, loss_s = _tc_quantize(
            xy_s, W1, b1r, gr, ber, W2, b2r, cbT, cbT_bf, ks)
        qs.append(_sc_gather(codebook, idx_s.reshape(slab)))
        idxs.append(idx_s)
        losses.append(loss_s)

    q = jnp.concatenate(qs, axis=0)
    idx2d = jnp.concatenate(idxs, axis=0)
    loss = functools.reduce(jnp.add, losses).reshape(())
    return (q, idx2d, loss)


# speedup vs baseline: 1.2021x; 1.2021x over previous
import functools

import jax
import jax.numpy as jnp
from jax import lax
from jax.experimental import pallas as pl
from jax.experimental.pallas import tpu as pltpu
from jax.experimental.pallas import tpu_sc as plsc

_N = 65536
_H = 64
_D = 64
_K = 1024
_EPS = 1e-5

_SLABS = 1
_BN = 4096
_SC_CORES = 2
_SC_SUBCORES = 16
_NW = _SC_CORES * _SC_SUBCORES
_CHUNK = 512


def _tc_body(xy_ref, w1_ref, b1_ref, g_ref, be_ref, w2_ref, b2_ref, cbt_ref,
             cbtbf_ref, ks_ref, idx_ref, loss_ref, c2_ref):
    i = pl.program_id(0)

    @pl.when(i == 0)
    def _precompute():
        cbt_full = cbt_ref[...]
        c2_ref[...] = jnp.sum(cbt_full * cbt_full, axis=0, keepdims=True)

    bf = jnp.bfloat16
    xyf = xy_ref[...].astype(jnp.float32)
    nxy = xyf / 511.0 * 2.0 - 1.0
    h = jnp.dot(nxy.astype(bf), w1_ref[...].astype(bf),
                preferred_element_type=jnp.float32) + b1_ref[...]
    mu = jnp.mean(h, axis=-1, keepdims=True)
    var = jnp.mean((h - mu) ** 2, axis=-1, keepdims=True)
    h = (h - mu) / jnp.sqrt(var + _EPS) * g_ref[...] + be_ref[...]
    h = jnp.maximum(h, 0.0)
    z = jnp.dot(h.astype(bf), w2_ref[...].astype(bf),
                preferred_element_type=jnp.float32) + b2_ref[...]

    z2 = jnp.sum(z * z, axis=1, keepdims=True)
    c2 = c2_ref[...]
    zc2 = jnp.dot((2.0 * z).astype(bf), cbtbf_ref[...],
                  preferred_element_type=jnp.float32)
    d = z2 - zc2 + c2
    dmin = jnp.min(d, axis=1, keepdims=True)
    kf = jnp.where(d == dmin, ks_ref[...], float(_K))
    imin = jnp.min(kf, axis=1, keepdims=True).astype(jnp.int32)
    idx_ref[...] = imin

    @pl.when(i == 0)
    def _init():
        loss_ref[...] = jnp.zeros((1, 1), jnp.float32)

    loss_ref[...] += jnp.sum(dmin).reshape(1, 1)

    @pl.when(i == pl.num_programs(0) - 1)
    def _finish():
        loss_ref[...] = loss_ref[...] * (1.25 / (_N * _D))


def _tc_quantize(xy, W1, b1, gamma, beta, W2, b2, cbT, cbT_bf, ks):
    n = xy.shape[0]
    rep = lambda i: (0, 0)
    return pl.pallas_call(
        _tc_body,
        grid=(n // _BN,),
        in_specs=[
            pl.BlockSpec((_BN, 2), lambda i: (i, 0)),
            pl.BlockSpec((2, _H), rep),
            pl.BlockSpec((1, _H), rep),
            pl.BlockSpec((1, _H), rep),
            pl.BlockSpec((1, _H), rep),
            pl.BlockSpec((_H, _D), rep),
            pl.BlockSpec((1, _D), rep),
            pl.BlockSpec((_D, _K), rep),
            pl.BlockSpec((_D, _K), rep),
            pl.BlockSpec((1, _K), rep),
        ],
        out_specs=[
            pl.BlockSpec((_BN, 1), lambda i: (i, 0)),
            pl.BlockSpec((1, 1), rep),
        ],
        out_shape=[
            jax.ShapeDtypeStruct((n, 1), jnp.int32),
            jax.ShapeDtypeStruct((1, 1), jnp.float32),
        ],
        scratch_shapes=[pltpu.VMEM((1, _K), jnp.float32)],
        compiler_params=pltpu.CompilerParams(
            dimension_semantics=("arbitrary",)),
    )(xy, W1, b1, gamma, beta, W2, b2, cbT, cbT_bf, ks)


def _sc_gather(codebook, idx_flat):
    n = idx_flat.shape[0]
    rows_per_w = n // _NW
    mesh = plsc.VectorSubcoreMesh(core_axis_name="c", subcore_axis_name="s")

    @functools.partial(
        pl.kernel,
        mesh=mesh,
        out_type=jax.ShapeDtypeStruct((n, _D), jnp.float32),
        scratch_types=[
            pltpu.VMEM((_K, _D), jnp.float32),
            pltpu.VMEM_SHARED((_K, _D), jnp.float32),
            pltpu.VMEM((_CHUNK,), jnp.int32),
            pltpu.VMEM((_CHUNK, _D), jnp.float32),
            pltpu.SemaphoreType.DMA,
        ],
        compiler_params=pltpu.CompilerParams(use_tc_tiling_on_sc=False),
    )
    def gather_kernel(cb_hbm, idx_hbm, out_hbm, tmp_v, cb_sh, idx_v, rows_v,
                      sem):
        sid = lax.axis_index("s")
        wid = sid * _SC_CORES + lax.axis_index("c")

        @pl.when(sid == 0)
        def _stage():
            pltpu.sync_copy(cb_hbm, tmp_v)
            pltpu.sync_copy(tmp_v, cb_sh)

        plsc.subcore_barrier()
        base = wid * rows_per_w
        for c in range(rows_per_w // _CHUNK):
            off = base + c * _CHUNK
            pltpu.sync_copy(idx_hbm.at[pl.ds(off, _CHUNK)], idx_v)
            pltpu.async_copy(cb_sh.at[idx_v], rows_v, sem).wait()
            pltpu.sync_copy(rows_v, out_hbm.at[pl.ds(off, _CHUNK)])

    return gather_kernel(codebook, idx_flat)


def kernel(xy, W1, b1, gamma, beta, W2, b2, codebook):
    cbT = codebook.T
    cbT_bf = cbT.astype(jnp.bfloat16)
    ks = jnp.arange(_K, dtype=jnp.float32).reshape(1, _K)
    b1r, gr, ber = (b1.reshape(1, _H), gamma.reshape(1, _H),
                    beta.reshape(1, _H))
    b2r = b2.reshape(1, _D)

    slab = _N // _SLABS
    qs, idxs, losses = [], [], []
    for s in range(_SLABS):
        xy_s = lax.slice_in_dim(xy, s * slab, (s + 1) * slab, axis=0)
        idx_s, loss_s = _tc_quantize(
            xy_s, W1, b1r, gr, ber, W2, b2r, cbT, cbT_bf, ks)
        qs.append(_sc_gather(codebook, idx_s.reshape(slab)))
        idxs.append(idx_s)
        losses.append(loss_s)

    if _SLABS == 1:
        return (qs[0], idxs[0], losses[0].reshape(()))
    q = jnp.concatenate(qs, axis=0)
    idx2d = jnp.concatenate(idxs, axis=0)
    loss = functools.reduce(jnp.add, losses).reshape(())
    return (q, idx2d, loss)


# speedup vs baseline: 1.2675x; 1.0544x over previous
import functools

import jax
import jax.numpy as jnp
from jax import lax
from jax.experimental import pallas as pl
from jax.experimental.pallas import tpu as pltpu
from jax.experimental.pallas import tpu_sc as plsc

_N = 65536
_H = 64
_D = 64
_K = 1024
_EPS = 1e-5

_SLABS = 1
_BN = 4096
_SC_CORES = 2
_SC_SUBCORES = 16
_NW = _SC_CORES * _SC_SUBCORES
_CHUNK = 512


def _tc_body(xy_ref, w1_ref, b1_ref, g_ref, be_ref, w2_ref, b2_ref, cbt_ref,
             cbtbf_ref, ks_ref, idx_ref, idxp_ref, loss_ref, c2_ref):
    i = pl.program_id(0)

    @pl.when(i == 0)
    def _precompute():
        cbt_full = cbt_ref[...]
        c2_ref[...] = jnp.sum(cbt_full * cbt_full, axis=0, keepdims=True)

    bf = jnp.bfloat16
    xyf = xy_ref[...].astype(jnp.float32)
    nxy = xyf / 511.0 * 2.0 - 1.0
    h = jnp.dot(nxy.astype(bf), w1_ref[...].astype(bf),
                preferred_element_type=jnp.float32) + b1_ref[...]
    mu = jnp.mean(h, axis=-1, keepdims=True)
    var = jnp.mean((h - mu) ** 2, axis=-1, keepdims=True)
    h = (h - mu) / jnp.sqrt(var + _EPS) * g_ref[...] + be_ref[...]
    h = jnp.maximum(h, 0.0)
    z = jnp.dot(h.astype(bf), w2_ref[...].astype(bf),
                preferred_element_type=jnp.float32) + b2_ref[...]

    z2 = jnp.sum(z * z, axis=1, keepdims=True)
    c2 = c2_ref[...]
    zc2 = jnp.dot((2.0 * z).astype(bf), cbtbf_ref[...],
                  preferred_element_type=jnp.float32)
    d = z2 - zc2 + c2
    dmin = jnp.min(d, axis=1, keepdims=True)
    kf = jnp.where(d == dmin, ks_ref[...], float(_K))
    imin = jnp.min(kf, axis=1, keepdims=True).astype(jnp.int32)
    idx_ref[...] = imin
    idxp_ref[...] = imin.reshape(_BN // 128, 128)

    @pl.when(i == 0)
    def _init():
        loss_ref[...] = jnp.zeros((1, 1), jnp.float32)

    loss_ref[...] += jnp.sum(dmin).reshape(1, 1)

    @pl.when(i == pl.num_programs(0) - 1)
    def _finish():
        loss_ref[...] = loss_ref[...] * (1.25 / (_N * _D))


def _tc_quantize(xy, W1, b1, gamma, beta, W2, b2, cbT, cbT_bf, ks):
    n = xy.shape[0]
    rep = lambda i: (0, 0)
    return pl.pallas_call(
        _tc_body,
        grid=(n // _BN,),
        in_specs=[
            pl.BlockSpec((_BN, 2), lambda i: (i, 0)),
            pl.BlockSpec((2, _H), rep),
            pl.BlockSpec((1, _H), rep),
            pl.BlockSpec((1, _H), rep),
            pl.BlockSpec((1, _H), rep),
            pl.BlockSpec((_H, _D), rep),
            pl.BlockSpec((1, _D), rep),
            pl.BlockSpec((_D, _K), rep),
            pl.BlockSpec((_D, _K), rep),
            pl.BlockSpec((1, _K), rep),
        ],
        out_specs=[
            pl.BlockSpec((_BN, 1), lambda i: (i, 0)),
            pl.BlockSpec((_BN // 128, 128), lambda i: (i, 0)),
            pl.BlockSpec((1, 1), rep),
        ],
        out_shape=[
            jax.ShapeDtypeStruct((n, 1), jnp.int32),
            jax.ShapeDtypeStruct((n // 128, 128), jnp.int32),
            jax.ShapeDtypeStruct((1, 1), jnp.float32),
        ],
        scratch_shapes=[pltpu.VMEM((1, _K), jnp.float32)],
        compiler_params=pltpu.CompilerParams(
            dimension_semantics=("arbitrary",)),
    )(xy, W1, b1, gamma, beta, W2, b2, cbT, cbT_bf, ks)


def _sc_gather(codebook, idx_flat):
    n = idx_flat.shape[0]
    rows_per_w = n // _NW
    mesh = plsc.VectorSubcoreMesh(core_axis_name="c", subcore_axis_name="s")

    @functools.partial(
        pl.kernel,
        mesh=mesh,
        out_type=jax.ShapeDtypeStruct((n, _D), jnp.float32),
        scratch_types=[
            pltpu.VMEM((_K, _D), jnp.float32),
            pltpu.VMEM_SHARED((_K, _D), jnp.float32),
            pltpu.VMEM((_CHUNK,), jnp.int32),
            pltpu.VMEM((_CHUNK, _D), jnp.float32),
            pltpu.SemaphoreType.DMA,
        ],
        compiler_params=pltpu.CompilerParams(use_tc_tiling_on_sc=False),
    )
    def gather_kernel(cb_hbm, idx_hbm, out_hbm, tmp_v, cb_sh, idx_v, rows_v,
                      sem):
        sid = lax.axis_index("s")
        wid = sid * _SC_CORES + lax.axis_index("c")

        @pl.when(sid == 0)
        def _stage():
            pltpu.sync_copy(cb_hbm, tmp_v)
            pltpu.sync_copy(tmp_v, cb_sh)

        plsc.subcore_barrier()
        base = wid * rows_per_w
        for c in range(rows_per_w // _CHUNK):
            off = base + c * _CHUNK
            pltpu.sync_copy(idx_hbm.at[pl.ds(off, _CHUNK)], idx_v)
            pltpu.async_copy(cb_sh.at[idx_v], rows_v, sem).wait()
            pltpu.sync_copy(rows_v, out_hbm.at[pl.ds(off, _CHUNK)])

    return gather_kernel(codebook, idx_flat)


def kernel(xy, W1, b1, gamma, beta, W2, b2, codebook):
    cbT = codebook.T
    cbT_bf = cbT.astype(jnp.bfloat16)
    ks = jnp.arange(_K, dtype=jnp.float32).reshape(1, _K)
    b1r, gr, ber = (b1.reshape(1, _H), gamma.reshape(1, _H),
                    beta.reshape(1, _H))
    b2r = b2.reshape(1, _D)

    slab = _N // _SLABS
    qs, idxs, losses = [], [], []
    for s in range(_SLABS):
        xy_s = lax.slice_in_dim(xy, s * slab, (s + 1) * slab, axis=0)
        idx_s, idxp_s, loss_s = _tc_quantize(
            xy_s, W1, b1r, gr, ber, W2, b2r, cbT, cbT_bf, ks)
        qs.append(_sc_gather(codebook, idxp_s.reshape(slab)))
        idxs.append(idx_s)
        losses.append(loss_s)

    if _SLABS == 1:
        return (qs[0], idxs[0], losses[0].reshape(()))
    q = jnp.concatenate(qs, axis=0)
    idx2d = jnp.concatenate(idxs, axis=0)
    loss = functools.reduce(jnp.add, losses).reshape(())
    return (q, idx2d, loss)
